# Initial kernel scaffold; baseline (speedup 1.0000x reference)
import jax
import jax.numpy as jnp
from jax.experimental import pallas as pl


def kernel(x, edge_index, batch, W1, a_src1, a_dst1, b1, W2, a_src2, a_dst2, b2, fc_W, fc_b):
    raise NotImplementedError("write your pallas kernel here")


# speedup vs baseline: 23.9479x; 23.9479x over previous
import functools

import jax
import jax.numpy as jnp
from jax import lax
from jax.experimental import pallas as pl
from jax.experimental.pallas import tpu as pltpu
from jax.experimental.pallas import tpu_sc as plsc

N = 10000
E = 320000
D = 128
NCLASS = 16
NEG = 0.2

NW = 32
K = 128
NB = 81
EPAD = NW * NB * K
NPAD = 10240
RB = 256
SLICE = NPAD // 16


def _tc_head(xp, W, a2):
    def kfn(x_ref, W_ref, a2_ref, h_ref, al_ref):
        h = jnp.dot(x_ref[...], W_ref[...], preferred_element_type=jnp.float32)
        h_ref[...] = h
        al_ref[...] = jnp.dot(h, a2_ref[...], preferred_element_type=jnp.float32)

    return pl.pallas_call(
        kfn,
        grid=(NPAD // RB,),
        in_specs=[
            pl.BlockSpec((RB, D), lambda i: (i, 0)),
            pl.BlockSpec((D, D), lambda i: (0, 0)),
            pl.BlockSpec((D, 8), lambda i: (0, 0)),
        ],
        out_specs=[
            pl.BlockSpec((RB, D), lambda i: (i, 0)),
            pl.BlockSpec((RB, 8), lambda i: (i, 0)),
        ],
        out_shape=[
            jax.ShapeDtypeStruct((NPAD, D), jnp.float32),
            jax.ShapeDtypeStruct((NPAD, 8), jnp.float32),
        ],
    )(xp, W, a2)


def _elu(v):
    return jnp.where(v > 0, v, jnp.exp(jnp.minimum(v, 0.0)) - 1.0)


def _tc_mid(U, S, b, W, a2):
    def kfn(U_ref, S_ref, b_ref, W_ref, a2_ref, h_ref, al_ref):
        Us = U_ref[0] + U_ref[1]
        ss = S_ref[0] + S_ref[1]
        xb = Us / (ss[:, None] + 1e-16) + b_ref[...]
        xb = _elu(xb)
        h = jnp.dot(xb, W_ref[...], preferred_element_type=jnp.float32)
        h_ref[...] = h
        al_ref[...] = jnp.dot(h, a2_ref[...], preferred_element_type=jnp.float32)

    return pl.pallas_call(
        kfn,
        grid=(NPAD // RB,),
        in_specs=[
            pl.BlockSpec((2, RB, D), lambda i: (0, i, 0)),
            pl.BlockSpec((2, RB), lambda i: (0, i)),
            pl.BlockSpec((1, D), lambda i: (0, 0)),
            pl.BlockSpec((D, D), lambda i: (0, 0)),
            pl.BlockSpec((D, 8), lambda i: (0, 0)),
        ],
        out_specs=[
            pl.BlockSpec((RB, D), lambda i: (i, 0)),
            pl.BlockSpec((RB, 8), lambda i: (i, 0)),
        ],
        out_shape=[
            jax.ShapeDtypeStruct((NPAD, D), jnp.float32),
            jax.ShapeDtypeStruct((NPAD, 8), jnp.float32),
        ],
    )(U, S, b, W, a2)


def _tc_tail(U, S, b, fcW, fcb):
    def kfn(U_ref, S_ref, b_ref, fcW_ref, fcb_ref, o_ref):
        Us = U_ref[0] + U_ref[1]
        ss = S_ref[0] + S_ref[1]
        hb = Us / (ss[:, None] + 1e-16) + b_ref[...]
        hb = _elu(hb)
        logits = jnp.dot(hb, fcW_ref[...], preferred_element_type=jnp.float32)
        logits = logits + fcb_ref[...]
        m = jnp.max(logits, axis=1, keepdims=True)
        lse = jnp.log(jnp.sum(jnp.exp(logits - m), axis=1, keepdims=True)) + m
        o_ref[...] = logits - lse

    return pl.pallas_call(
        kfn,
        grid=(NPAD // RB,),
        in_specs=[
            pl.BlockSpec((2, RB, D), lambda i: (0, i, 0)),
            pl.BlockSpec((2, RB), lambda i: (0, i)),
            pl.BlockSpec((1, D), lambda i: (0, 0)),
            pl.BlockSpec((D, NCLASS), lambda i: (0, 0)),
            pl.BlockSpec((1, NCLASS), lambda i: (0, 0)),
        ],
        out_specs=pl.BlockSpec((RB, NCLASS), lambda i: (i, 0)),
        out_shape=jax.ShapeDtypeStruct((NPAD, NCLASS), jnp.float32),
    )(U, S, b, fcW, fcb)


def _sc_body(h_hbm, asrc_hbm, adst_hbm, src_hbm, dst_hbm, U_hbm, S_hbm,
             asrc_v, adst_v, src_b, dst_b, rows, p_v, accA, accS, gsem):
    c = lax.axis_index("c")
    sid = lax.axis_index("s")
    wid = c * 16 + sid

    pltpu.sync_copy(asrc_hbm, asrc_v)
    pltpu.sync_copy(adst_hbm, adst_v)

    z = jnp.zeros((16,), jnp.float32)

    def zrow(r, carry):
        for cc in range(8):
            rows[r, pl.ds(cc * 16, 16)] = z
        return carry

    lax.fori_loop(0, K, zrow, 0)
    for i in range(8):
        p_v[pl.ds(i * 16, 16)] = z
    base = sid * SLICE
    for blk in range(SLICE // K):
        pltpu.sync_copy(rows, accA.at[pl.ds(base + blk * K, K)])
        pltpu.sync_copy(p_v, accS.at[pl.ds(base + blk * K, K)])
    plsc.subcore_barrier()

    def block_body(j, carry):
        pltpu.sync_copy(src_hbm.at[wid, j], src_b)
        pltpu.sync_copy(dst_hbm.at[wid, j], dst_b)
        gcp = pltpu.async_copy(h_hbm.at[src_b], rows, gsem)
        for i in range(8):
            sidx = src_b[pl.ds(i * 16, 16)]
            didx = dst_b[pl.ds(i * 16, 16)]
            e = plsc.load_gather(asrc_v, [sidx]) + plsc.load_gather(adst_v, [didx])
            e = jnp.where(e >= 0, e, NEG * e)
            p_v[pl.ds(i * 16, 16)] = jnp.exp(e)
        gcp.wait()

        def scale_grp(g, carry2):
            pvec = p_v[pl.ds(g * 16, 16)]
            for rr in range(16):
                pr = pvec[rr]
                r = g * 16 + rr
                for cc in range(8):
                    rows[r, pl.ds(cc * 16, 16)] = rows[r, pl.ds(cc * 16, 16)] * pr
            return carry2

        lax.fori_loop(0, K // 16, scale_grp, 0)
        pltpu.sync_copy(rows, accA.at[dst_b], add=True)
        pltpu.sync_copy(p_v, accS.at[dst_b], add=True)
        return carry

    lax.fori_loop(0, NB, block_body, 0)
    plsc.subcore_barrier()

    pltpu.sync_copy(accA.at[pl.ds(base, SLICE)], U_hbm.at[c, pl.ds(base, SLICE)])
    pltpu.sync_copy(accS.at[pl.ds(base, SLICE)], S_hbm.at[c, pl.ds(base, SLICE)])


def _sc_edge(h, asrc, adst, src_r, dst_r):
    mesh = plsc.VectorSubcoreMesh(core_axis_name="c", subcore_axis_name="s")
    fn = pl.kernel(
        _sc_body,
        out_type=[
            jax.ShapeDtypeStruct((2, NPAD, D), jnp.float32),
            jax.ShapeDtypeStruct((2, NPAD), jnp.float32),
        ],
        mesh=mesh,
        compiler_params=pltpu.CompilerParams(needs_layout_passes=False),
        scratch_types=[
            pltpu.VMEM((NPAD,), jnp.float32),
            pltpu.VMEM((NPAD,), jnp.float32),
            pltpu.VMEM((K,), jnp.int32),
            pltpu.VMEM((K,), jnp.int32),
            pltpu.VMEM((K, D), jnp.float32),
            pltpu.VMEM((K,), jnp.float32),
            pltpu.VMEM_SHARED((NPAD, D), jnp.float32),
            pltpu.VMEM_SHARED((NPAD,), jnp.float32),
            pltpu.SemaphoreType.DMA,
        ],
    )
    return fn(h, asrc, adst, src_r, dst_r)


def kernel(x, edge_index, batch, W1, a_src1, a_dst1, b1, W2, a_src2, a_dst2, b2, fc_W, fc_b):
    loop = jnp.arange(N, dtype=jnp.int32)
    pad = jnp.full((EPAD - E - N,), N, dtype=jnp.int32)
    src_r = jnp.concatenate([edge_index[0], loop, pad]).reshape(NW, NB, K)
    dst_r = jnp.concatenate([edge_index[1], loop, pad]).reshape(NW, NB, K)

    xp = jnp.zeros((NPAD, D), jnp.float32).at[:N].set(x)
    a2_1 = jnp.zeros((D, 8), jnp.float32).at[:, 0].set(a_src1).at[:, 1].set(a_dst1)
    a2_2 = jnp.zeros((D, 8), jnp.float32).at[:, 0].set(a_src2).at[:, 1].set(a_dst2)

    h1, al1 = _tc_head(xp, W1, a2_1)
    U1, S1 = _sc_edge(h1, al1[:, 0], al1[:, 1], src_r, dst_r)
    h2, al2 = _tc_mid(U1, S1, b1.reshape(1, D), W2, a2_2)
    U2, S2 = _sc_edge(h2, al2[:, 0], al2[:, 1], src_r, dst_r)
    out = _tc_tail(U2, S2, b2.reshape(1, D), fc_W, fc_b.reshape(1, NCLASS))
    return out[:N]
